# baseline (speedup 1.0000x reference)
import functools

import jax
import jax.numpy as jnp
from jax import lax
from jax.experimental import pallas as pl
from jax.experimental.pallas import tpu as pltpu
from jax.experimental.pallas import tpu_sc as plsc

D = 1024
H = 73
R = 4 * 2048
NC, NS = 2, 16
NW = NC * NS
RPW = R // NW
CH = 16
NCHUNK = RPW // CH
HB = CH * H
NEG_INF = float("-inf")

_mesh = plsc.VectorSubcoreMesh(
    core_axis_name="c", subcore_axis_name="s", num_cores=NC, num_subcores=NS
)


@functools.partial(
    pl.kernel,
    out_type=jax.ShapeDtypeStruct((R * D,), jnp.float32),
    mesh=_mesh,
    scratch_types=[
        pltpu.VMEM((H * D,), jnp.float32),
        pltpu.VMEM((CH * D,), jnp.float32),
        pltpu.VMEM((HB + 16,), jnp.float32),
    ],
    compiler_params=pltpu.CompilerParams(needs_layout_passes=False),
)
def _sc_add_pe(x_hbm, hour_hbm, pe_hbm, out_hbm, pe_v, xb_v, hb_v):
    wid = lax.axis_index("s") * NC + lax.axis_index("c")
    row0 = wid * RPW
    pltpu.sync_copy(pe_hbm, pe_v)
    io = lax.broadcasted_iota(jnp.int32, (16,), 0)

    def chunk_body(c, carry):
        base = row0 + c * CH
        pltpu.sync_copy(x_hbm.at[pl.ds(base * D, CH * D)], xb_v)
        pltpu.sync_copy(hour_hbm.at[pl.ds(base * H, HB)], hb_v.at[pl.ds(0, HB)])

        def row_body(r, rcarry):
            hb = r * H
            v0 = hb_v[pl.ds(hb, 16)]
            v1 = hb_v[pl.ds(hb + 16, 16)]
            v2 = hb_v[pl.ds(hb + 32, 16)]
            v3 = hb_v[pl.ds(hb + 48, 16)]
            v4 = hb_v[pl.ds(hb + 64, 16)]
            v4 = jnp.where(io < 9, v4, NEG_INF)
            mm = jnp.maximum(
                jnp.maximum(jnp.maximum(v0, v1), jnp.maximum(v2, v3)), v4
            )
            g = jnp.max(mm)
            big = jnp.int32(1 << 20)
            c0 = jnp.where(v0 == g, io, big)
            c1 = jnp.where(v1 == g, io + 16, big)
            c2 = jnp.where(v2 == g, io + 32, big)
            c3 = jnp.where(v3 == g, io + 48, big)
            c4 = jnp.where(v4 == g, io + 64, big)
            idx = jnp.min(
                jnp.minimum(
                    jnp.minimum(jnp.minimum(c0, c1), jnp.minimum(c2, c3)), c4
                )
            )
            pb = idx * D
            xrow = r * D
            for j in range(D // 16):
                pv = pe_v[pl.ds(pb + j * 16, 16)]
                plsc.addupdate(xb_v.at[pl.ds(xrow + j * 16, 16)], pv)
            return rcarry

        lax.fori_loop(0, CH, row_body, 0)
        pltpu.sync_copy(xb_v, out_hbm.at[pl.ds(base * D, CH * D)])
        return carry

    lax.fori_loop(0, NCHUNK, chunk_body, 0)


def kernel(x, hour_onehot, pe):
    out = _sc_add_pe(
        x.reshape(-1), hour_onehot.reshape(-1), pe.reshape(-1)
    )
    return out.reshape(x.shape)


# speedup vs baseline: 1.1089x; 1.1089x over previous
import functools

import jax
import jax.numpy as jnp
from jax import lax
from jax.experimental import pallas as pl
from jax.experimental.pallas import tpu as pltpu
from jax.experimental.pallas import tpu_sc as plsc

D = 1024
H = 73
R = 4 * 2048
NC, NS = 2, 16
NW = NC * NS
RPW = R // NW
CH = 16
NCHUNK = RPW // CH
HB = CH * H
NEG_INF = float("-inf")

_mesh = plsc.VectorSubcoreMesh(
    core_axis_name="c", subcore_axis_name="s", num_cores=NC, num_subcores=NS
)


@functools.partial(
    pl.kernel,
    out_type=jax.ShapeDtypeStruct((R * D,), jnp.float32),
    mesh=_mesh,
    scratch_types=[
        pltpu.VMEM((H * D,), jnp.float32),
        pltpu.VMEM((2 * CH * D,), jnp.float32),
        pltpu.VMEM((2 * (HB + 16),), jnp.float32),
        pltpu.SemaphoreType.DMA,
        pltpu.SemaphoreType.DMA,
        pltpu.SemaphoreType.DMA,
        pltpu.SemaphoreType.DMA,
        pltpu.SemaphoreType.DMA,
    ],
    compiler_params=pltpu.CompilerParams(needs_layout_passes=False),
)
def _sc_add_pe(x_hbm, hour_hbm, pe_hbm, out_hbm, pe_v, xb_v, hb_v,
               pe_sem, in0_sem, in1_sem, out0_sem, out1_sem):
    wid = lax.axis_index("s") * NC + lax.axis_index("c")
    row0 = wid * RPW
    io = lax.broadcasted_iota(jnp.int32, (16,), 0)
    in_sems = (in0_sem, in1_sem)
    out_sems = (out0_sem, out1_sem)

    def in_copies(c, p):
        base = row0 + c * CH
        return (
            pltpu.make_async_copy(
                x_hbm.at[pl.ds(base * D, CH * D)], xb_v.at[pl.ds(p * CH * D, CH * D)], in_sems[p]),
            pltpu.make_async_copy(
                hour_hbm.at[pl.ds(base * H, HB)],
                hb_v.at[pl.ds(p * (HB + 16), HB)], in_sems[p]),
        )

    def out_copy(c, p):
        base = row0 + c * CH
        return pltpu.make_async_copy(
            xb_v.at[pl.ds(p * CH * D, CH * D)], out_hbm.at[pl.ds(base * D, CH * D)], out_sems[p])

    def compute(p):
        def row_body(r, rcarry):
            hb = r * H
            v0 = hb_v[pl.ds(p * (HB + 16) + hb, 16)]
            v1 = hb_v[pl.ds(p * (HB + 16) + hb + 16, 16)]
            v2 = hb_v[pl.ds(p * (HB + 16) + hb + 32, 16)]
            v3 = hb_v[pl.ds(p * (HB + 16) + hb + 48, 16)]
            v4 = hb_v[pl.ds(p * (HB + 16) + hb + 64, 16)]
            v4 = jnp.where(io < 9, v4, NEG_INF)
            mm = jnp.maximum(
                jnp.maximum(jnp.maximum(v0, v1), jnp.maximum(v2, v3)), v4
            )
            g = jnp.max(mm)
            big = jnp.int32(1 << 20)
            c0 = jnp.where(v0 == g, io, big)
            c1 = jnp.where(v1 == g, io + 16, big)
            c2 = jnp.where(v2 == g, io + 32, big)
            c3 = jnp.where(v3 == g, io + 48, big)
            c4 = jnp.where(v4 == g, io + 64, big)
            idx = jnp.min(
                jnp.minimum(
                    jnp.minimum(jnp.minimum(c0, c1), jnp.minimum(c2, c3)), c4
                )
            )
            pb = idx * D
            xrow = r * D
            for j in range(D // 16):
                pv = pe_v[pl.ds(pb + j * 16, 16)]
                plsc.addupdate(xb_v.at[pl.ds(p * CH * D + xrow + j * 16, 16)], pv)
            return rcarry

        lax.fori_loop(0, CH, row_body, 0)

    pltpu.async_copy(pe_hbm, pe_v, pe_sem)
    for a, b in (in_copies(0, 0), in_copies(1, 1)):
        a.start()
        b.start()
    pltpu.make_async_copy(pe_hbm, pe_v, pe_sem).wait()

    for c in range(NCHUNK):
        p = c & 1
        q = p ^ 1
        if c + 1 < NCHUNK:
            if c >= 1:
                out_copy(c - 1, q).wait()
            if c + 1 >= 2:
                for cp in in_copies(c + 1, q):
                    cp.start()
        for cp in in_copies(c, p):
            cp.wait()
        compute(p)
        out_copy(c, p).start()

    out_copy(NCHUNK - 2, 0).wait()
    out_copy(NCHUNK - 1, 1).wait()


def kernel(x, hour_onehot, pe):
    out = _sc_add_pe(
        x.reshape(-1), hour_onehot.reshape(-1), pe.reshape(-1)
    )
    return out.reshape(x.shape)


# speedup vs baseline: 1.7832x; 1.6081x over previous
import functools

import jax
import jax.numpy as jnp
from jax import lax
from jax.experimental import pallas as pl
from jax.experimental.pallas import tpu as pltpu
from jax.experimental.pallas import tpu_sc as plsc

D = 1024
H = 73
R = 4 * 2048
NC, NS = 2, 16
NW = NC * NS
RPW = R // NW
CH = 16
NCHUNK = RPW // CH
HSTR = R
HWB = H * RPW

_mesh = plsc.VectorSubcoreMesh(
    core_axis_name="c", subcore_axis_name="s", num_cores=NC, num_subcores=NS
)


@functools.partial(
    pl.kernel,
    out_type=jax.ShapeDtypeStruct((R * D,), jnp.float32),
    mesh=_mesh,
    scratch_types=[
        pltpu.VMEM((H * D,), jnp.float32),
        pltpu.VMEM((2 * CH * D,), jnp.float32),
        pltpu.VMEM((HWB,), jnp.float32),
        pltpu.VMEM((RPW,), jnp.int32),
        pltpu.SemaphoreType.DMA,
        pltpu.SemaphoreType.DMA,
        pltpu.SemaphoreType.DMA,
        pltpu.SemaphoreType.DMA,
        pltpu.SemaphoreType.DMA,
        pltpu.SemaphoreType.DMA,
    ],
    compiler_params=pltpu.CompilerParams(needs_layout_passes=False),
)
def _sc_add_pe(x_hbm, hour_hbm, pe_hbm, out_hbm, pe_v, xb_v, hbuf, idx_v,
               pe_sem, hr_sem, in0_sem, in1_sem, out0_sem, out1_sem):
    wid = lax.axis_index("s") * NC + lax.axis_index("c")
    row0 = wid * RPW
    b = wid // 8
    lw = wid % 8
    io = lax.broadcasted_iota(jnp.int32, (16,), 0)
    in_sems = (in0_sem, in1_sem)
    out_sems = (out0_sem, out1_sem)

    def in_copy(c, p):
        base = row0 + c * CH
        return pltpu.make_async_copy(
            x_hbm.at[pl.ds(base * D, CH * D)],
            xb_v.at[pl.ds(p * CH * D, CH * D)], in_sems[p])

    def out_copy(c, p):
        base = row0 + c * CH
        return pltpu.make_async_copy(
            xb_v.at[pl.ds(p * CH * D, CH * D)],
            out_hbm.at[pl.ds(base * D, CH * D)], out_sems[p])

    pltpu.async_copy(pe_hbm, pe_v, pe_sem)

    def hour_dma(h, carry):
        src0 = h * HSTR + (lw * 2) * 512 + b * 128
        pltpu.async_copy(hour_hbm.at[pl.ds(src0, 128)],
                         hbuf.at[pl.ds(h * RPW, 128)], hr_sem)
        pltpu.async_copy(hour_hbm.at[pl.ds(src0 + 512, 128)],
                         hbuf.at[pl.ds(h * RPW + 128, 128)], hr_sem)
        return carry

    lax.fori_loop(0, H, hour_dma, 0)
    in_copy(0, 0).start()
    in_copy(1, 1).start()

    pltpu.make_async_copy(hour_hbm.at[pl.ds(0, HWB)], hbuf, hr_sem).wait()

    def amax_group(g, carry):
        col = g * 16
        best = hbuf[pl.ds(col, 16)]
        besti = io * 0
        for h in range(1, H):
            v = hbuf[pl.ds(h * RPW + col, 16)]
            upd = v > best
            besti = jnp.where(upd, h, besti)
            best = jnp.where(upd, v, best)
        idx_v[pl.ds(col, 16)] = besti * D
        return carry

    lax.fori_loop(0, RPW // 16, amax_group, 0)
    pltpu.make_async_copy(pe_hbm, pe_v, pe_sem).wait()

    def compute(c, p):
        xbase = p * CH * D

        def row_body(j, rcarry):
            ivec = idx_v[pl.ds(c * CH, 16)]
            pb = jnp.max(jnp.where(io == j, ivec, 0))
            tl = j // 8
            rr = j - tl * 8
            xoff = xbase + tl * 8192 + rr * 128
            for ct in range(8):
                for k in range(8):
                    pv = pe_v[pl.ds(pb + ct * 128 + k * 16, 16)]
                    plsc.addupdate(
                        xb_v.at[pl.ds(xoff + ct * 1024 + k * 16, 16)], pv)
            return rcarry

        lax.fori_loop(0, CH, row_body, 0)

    for c in range(NCHUNK):
        p = c & 1
        q = p ^ 1
        if c + 1 < NCHUNK:
            if c >= 1:
                out_copy(c - 1, q).wait()
            if c + 1 >= 2:
                in_copy(c + 1, q).start()
        in_copy(c, p).wait()
        compute(c, p)
        out_copy(c, p).start()

    out_copy(NCHUNK - 2, 0).wait()
    out_copy(NCHUNK - 1, 1).wait()


def kernel(x, hour_onehot, pe):
    xf = x.reshape(1024, 8, 8, 128).transpose(0, 2, 1, 3).reshape(-1)
    hf = hour_onehot.reshape(4, 16, 128, H).transpose(3, 1, 0, 2).reshape(-1)
    pf = pe.reshape(-1)
    of = _sc_add_pe(xf, hf, pf)
    return of.reshape(1024, 8, 8, 128).transpose(0, 2, 1, 3).reshape(x.shape)
